# baseline (speedup 1.0000x reference)
import functools

import jax
import jax.numpy as jnp
from jax import lax
from jax.experimental import pallas as pl
from jax.experimental.pallas import tpu as pltpu
from jax.experimental.pallas import tpu_sc as plsc

_INC, _INN, _OUTC, _OUTN, _MAXD, _N = 128, 256, 128, 64, 4, 1024
_NW = 32
_BR = 128
_HI = _INC // 2
_HB = _HI // 16
_NB = _INC // 16
_PU = _OUTN * _HI
_OCH = 4
_RING = 4


def _sc_pool(xu, wgw, axb, rows):
    units_per = rows * 2 // _NW
    mesh = plsc.VectorSubcoreMesh(core_axis_name="c", subcore_axis_name="s")

    @functools.partial(
        pl.kernel,
        mesh=mesh,
        compiler_params=pltpu.CompilerParams(
            needs_layout_passes=False, use_tc_tiling_on_sc=False),
        out_type=jax.ShapeDtypeStruct((rows * 2, _PU), jnp.float32),
        scratch_types=(
            [pltpu.VMEM((_HI, _INN + 1), jnp.float32) for _ in range(_RING)]
            + [pltpu.VMEM((_PU,), jnp.float32) for _ in range(_RING)]
            + [pltpu.VMEM((_OUTN * _NB * _MAXD * 16,), jnp.float32),
               pltpu.VMEM((_OUTN * _MAXD * 16,), jnp.int32)]
            + [pltpu.SemaphoreType.DMA for _ in range(2 * _RING)]
        ),
    )
    def k(x_hbm, wgw_hbm, axb_hbm, pooled_hbm,
          xb0, xb1, xb2, xb3, pv0, pv1, pv2, pv3, wgw_v, axb_v,
          sx0, sx1, sx2, sx3, sp0, sp1, sp2, sp3):
        xbs = (xb0, xb1, xb2, xb3)
        pvs = (pv0, pv1, pv2, pv3)
        sxs = (sx0, sx1, sx2, sx3)
        sps = (sp0, sp1, sp2, sp3)
        wid = lax.axis_index("s") * 2 + lax.axis_index("c")
        ubase = wid * units_per
        pltpu.sync_copy(wgw_hbm, wgw_v)
        pltpu.sync_copy(axb_hbm, axb_v)

        ibases = [lax.iota(jnp.int32, 16) + 16 * b for b in range(_HB)]

        def compute(xb, pv, h):
            def obody(oc, carry):
                for oo in range(_OCH):
                    o = oc * _OCH + oo
                    abase = pl.multiple_of(o * (_MAXD * 16), _MAXD * 16)
                    wbase = pl.multiple_of(o * (_NB * _MAXD * 16),
                                           _NB * _MAXD * 16)
                    pbase = pl.multiple_of(o * _HI, _HI)
                    kv = [axb_v[pl.ds(abase + 16 * d, 16)]
                          for d in range(_MAXD)]
                    for b in range(_HB):
                        bg = h * _HB + b
                        acc = None
                        for d in range(_MAXD):
                            wv = wgw_v[pl.ds(
                                wbase + (bg * _MAXD + d) * 16, 16)]
                            xg = plsc.load_gather(xb, [ibases[b], kv[d]])
                            t = wv * xg
                            acc = t if acc is None else acc + t
                        pv[pl.ds(pbase + b * 16, 16)] = acc
                return carry
            lax.fori_loop(0, _OUTN // _OCH, obody, 0)

        for s in range(_RING):
            pltpu.make_async_copy(
                x_hbm.at[ubase + s], xbs[s].at[:, pl.ds(0, _INN)],
                sxs[s]).start()

        def gbody(g, carry):
            u0 = ubase + _RING * g
            for s in range(_RING):
                u = u0 + s
                pltpu.make_async_copy(
                    x_hbm.at[u], xbs[s].at[:, pl.ds(0, _INN)],
                    sxs[s]).wait()

                @pl.when(g > 0)
                def _():
                    pltpu.make_async_copy(pvs[s], pooled_hbm.at[u],
                                          sps[s]).wait()
                compute(xbs[s], pvs[s], s & 1)
                pltpu.make_async_copy(pvs[s], pooled_hbm.at[u],
                                      sps[s]).start()

                @pl.when(g < units_per // _RING - 1)
                def _():
                    pltpu.make_async_copy(
                        x_hbm.at[u + _RING], xbs[s].at[:, pl.ds(0, _INN)],
                        sxs[s]).start()
            return carry

        lax.fori_loop(0, units_per // _RING, gbody, 0)
        for s in range(_RING):
            pltpu.make_async_copy(pvs[s], pooled_hbm.at[ubase], sps[s]).wait()

    return k(xu, wgw, axb)


def _post_block(p_ref, ctw_ref, b_ref, o_ref):
    ctw = ctw_ref[...]
    b = b_ref[...]
    for j in range(_BR):
        o_ref[j] = (
            lax.dot_general(
                ctw[:, :_HI], p_ref[j, 0], (((1,), (1,)), ((), ())),
                preferred_element_type=jnp.float32,
            )
            + lax.dot_general(
                ctw[:, _HI:], p_ref[j, 1], (((1,), (1,)), ((), ())),
                preferred_element_type=jnp.float32,
            )
            + b
        )


def kernel(x, weight, mask_weight, ct_w, ct_b, bias, A, mask):
    wm = (mask_weight * mask).reshape(_OUTN, _MAXD)
    wgw = ((wm[:, :, None] * jnp.take(weight.T, A, axis=0))
           .reshape(_OUTN, _MAXD, _NB, 16)
           .transpose(0, 2, 1, 3).reshape(-1))
    axb = jnp.broadcast_to(A.reshape(_OUTN * _MAXD, 1),
                           (_OUTN * _MAXD, 16)).astype(jnp.int32).reshape(-1)
    b2 = bias + ct_b[:, None]

    xu = x.reshape(_N * 2, _HI, _INN)
    pooled = _sc_pool(xu, wgw, axb, _N).reshape(_N, 2, _OUTN, _HI)

    grid = (_N // _BR,)
    return pl.pallas_call(
        _post_block,
        grid=grid,
        in_specs=[
            pl.BlockSpec((_BR, 2, _OUTN, _HI), lambda i: (i, 0, 0, 0)),
            pl.BlockSpec((_OUTC, _INC), lambda i: (0, 0)),
            pl.BlockSpec((_OUTC, _OUTN), lambda i: (0, 0)),
        ],
        out_specs=pl.BlockSpec((_BR, _OUTC, _OUTN), lambda i: (i, 0, 0)),
        out_shape=jax.ShapeDtypeStruct((_N, _OUTC, _OUTN), jnp.float32),
        compiler_params=pltpu.CompilerParams(
            dimension_semantics=("parallel",),
        ),
    )(pooled, ct_w, b2)


# speedup vs baseline: 2.0608x; 2.0608x over previous
import functools

import jax
import jax.numpy as jnp
from jax import lax
from jax.experimental import pallas as pl
from jax.experimental.pallas import tpu as pltpu
from jax.experimental.pallas import tpu_sc as plsc

_INC, _INN, _OUTC, _OUTN, _MAXD, _N = 128, 256, 128, 64, 4, 1024
_NW = 32
_BR = 128
_HI = _INC // 2
_HB = _HI // 16
_NB = _INC // 16
_PU = _OUTN * _HI
_OCH = 4
_RING = 4


def _sc_pool(xu, wgw, axb, rows):
    units_per = rows * 2 // _NW
    mesh = plsc.VectorSubcoreMesh(core_axis_name="c", subcore_axis_name="s")

    @functools.partial(
        pl.kernel,
        mesh=mesh,
        compiler_params=pltpu.CompilerParams(
            needs_layout_passes=False, use_tc_tiling_on_sc=False),
        out_type=jax.ShapeDtypeStruct((rows * 2, _PU), jnp.float32),
        scratch_types=(
            [pltpu.VMEM((_HI, _INN + 1), jnp.float32) for _ in range(_RING)]
            + [pltpu.VMEM((_PU,), jnp.float32) for _ in range(_RING)]
            + [pltpu.VMEM((_OUTN * _NB * _MAXD * 16,), jnp.float32),
               pltpu.VMEM((_OUTN * _MAXD * 16,), jnp.int32)]
            + [pltpu.SemaphoreType.DMA for _ in range(2 * _RING)]
        ),
    )
    def k(x_hbm, wgw_hbm, axb_hbm, pooled_hbm,
          xb0, xb1, xb2, xb3, pv0, pv1, pv2, pv3, wgw_v, axb_v,
          sx0, sx1, sx2, sx3, sp0, sp1, sp2, sp3):
        xbs = (xb0, xb1, xb2, xb3)
        pvs = (pv0, pv1, pv2, pv3)
        sxs = (sx0, sx1, sx2, sx3)
        sps = (sp0, sp1, sp2, sp3)
        wid = lax.axis_index("s") * 2 + lax.axis_index("c")
        ubase = wid * units_per
        pltpu.sync_copy(wgw_hbm, wgw_v)
        pltpu.sync_copy(axb_hbm, axb_v)

        ibases = [lax.iota(jnp.int32, 16) + 16 * b for b in range(_HB)]

        def compute(xb, pv, h):
            def obody(oc, carry):
                for oo in range(_OCH):
                    o = oc * _OCH + oo
                    abase = pl.multiple_of(o * (_MAXD * 16), _MAXD * 16)
                    wbase = pl.multiple_of(o * (_NB * _MAXD * 16),
                                           _NB * _MAXD * 16)
                    pbase = pl.multiple_of(o * _HI, _HI)
                    kv = [axb_v[pl.ds(abase + 16 * d, 16)]
                          for d in range(_MAXD)]
                    for b in range(_HB):
                        bg = h * _HB + b
                        acc = None
                        for d in range(_MAXD):
                            wv = wgw_v[pl.ds(
                                wbase + (bg * _MAXD + d) * 16, 16)]
                            xg = plsc.load_gather(xb, [ibases[b], kv[d]])
                            t = wv * xg
                            acc = t if acc is None else acc + t
                        pv[pl.ds(pbase + b * 16, 16)] = acc
                return carry
            lax.fori_loop(0, _OUTN // _OCH, obody, 0)

        for s in range(_RING):
            pltpu.make_async_copy(
                x_hbm.at[ubase + s], xbs[s].at[:, pl.ds(0, _INN)],
                sxs[s]).start()

        def gbody(g, carry):
            u0 = ubase + _RING * g
            for s in range(_RING):
                u = u0 + s
                pltpu.make_async_copy(
                    x_hbm.at[u], xbs[s].at[:, pl.ds(0, _INN)],
                    sxs[s]).wait()

                @pl.when(g > 0)
                def _():
                    pltpu.make_async_copy(pvs[s], pooled_hbm.at[u],
                                          sps[s]).wait()
                compute(xbs[s], pvs[s], s & 1)
                pltpu.make_async_copy(pvs[s], pooled_hbm.at[u],
                                      sps[s]).start()

                @pl.when(g < units_per // _RING - 1)
                def _():
                    pltpu.make_async_copy(
                        x_hbm.at[u + _RING], xbs[s].at[:, pl.ds(0, _INN)],
                        sxs[s]).start()
            return carry

        lax.fori_loop(0, units_per // _RING, gbody, 0)
        for s in range(_RING):
            pltpu.make_async_copy(pvs[s], pooled_hbm.at[ubase], sps[s]).wait()

    return k(xu, wgw, axb)


_RSC = 192
_BN1 = 64
_BR2 = 64


def _fgl_block(x_ref, w_ref, wm_ref, a_ref, ctw_ref, b_ref, out_ref):
    k_iota = lax.broadcasted_iota(jnp.int32, (_OUTN, _INN), 1)
    a = a_ref[...]
    wmv = wm_ref[...]
    ct = jnp.zeros((_OUTN, _INN), jnp.float32)
    for d in range(_MAXD):
        ct = ct + jnp.where(k_iota == a[:, d : d + 1], wmv[:, d : d + 1], 0.0)

    xw = x_ref[...] * w_ref[...][None, :, :]
    pooled = lax.dot_general(
        xw.reshape(_BN1 * _INC, _INN), ct,
        (((1,), (1,)), ((), ())),
        preferred_element_type=jnp.float32,
    ).reshape(_BN1, _INC, _OUTN)

    ctw = ctw_ref[...]
    b = b_ref[...]
    for j in range(_BN1):
        out_ref[j] = (
            lax.dot_general(
                ctw, pooled[j], (((1,), (0,)), ((), ())),
                preferred_element_type=jnp.float32,
            )
            + b
        )


def _post_block(p_ref, ctw_ref, b_ref, y_ref, o_ref):
    del y_ref
    ctw = ctw_ref[...]
    b = b_ref[...]
    for j in range(_BR2):
        o_ref[j] = (
            lax.dot_general(
                ctw[:, :_HI], p_ref[j, 0], (((1,), (1,)), ((), ())),
                preferred_element_type=jnp.float32,
            )
            + lax.dot_general(
                ctw[:, _HI:], p_ref[j, 1], (((1,), (1,)), ((), ())),
                preferred_element_type=jnp.float32,
            )
            + b
        )


def kernel(x, weight, mask_weight, ct_w, ct_b, bias, A, mask):
    wm = (mask_weight * mask).reshape(_OUTN, _MAXD)
    wgw = ((wm[:, :, None] * jnp.take(weight.T, A, axis=0))
           .reshape(_OUTN, _MAXD, _NB, 16)
           .transpose(0, 2, 1, 3).reshape(-1))
    axb = jnp.broadcast_to(A.reshape(_OUTN * _MAXD, 1),
                           (_OUTN * _MAXD, 16)).astype(jnp.int32).reshape(-1)
    b2 = bias + ct_b[:, None]

    xu = x.reshape(_N * 2, _HI, _INN)
    pooled = _sc_pool(xu, wgw, axb, _RSC).reshape(_RSC, 2, _OUTN, _HI)

    off = _RSC // _BN1
    y1 = pl.pallas_call(
        _fgl_block,
        grid=((_N - _RSC) // _BN1,),
        in_specs=[
            pl.BlockSpec((_BN1, _INC, _INN), lambda i: (i + off, 0, 0)),
            pl.BlockSpec((_INC, _INN), lambda i: (0, 0)),
            pl.BlockSpec((_OUTN, _MAXD), lambda i: (0, 0)),
            pl.BlockSpec((_OUTN, _MAXD), lambda i: (0, 0)),
            pl.BlockSpec((_OUTC, _INC), lambda i: (0, 0)),
            pl.BlockSpec((_OUTC, _OUTN), lambda i: (0, 0)),
        ],
        out_specs=pl.BlockSpec((_BN1, _OUTC, _OUTN), lambda i: (i + off, 0, 0)),
        out_shape=jax.ShapeDtypeStruct((_N, _OUTC, _OUTN), jnp.float32),
        compiler_params=pltpu.CompilerParams(
            dimension_semantics=("arbitrary",),
        ),
    )(x, weight, wm, A, ct_w, b2)

    return pl.pallas_call(
        _post_block,
        grid=(_RSC // _BR2,),
        in_specs=[
            pl.BlockSpec((_BR2, 2, _OUTN, _HI), lambda i: (i, 0, 0, 0)),
            pl.BlockSpec((_OUTC, _INC), lambda i: (0, 0)),
            pl.BlockSpec((_OUTC, _OUTN), lambda i: (0, 0)),
            pl.BlockSpec((_BR2, _OUTC, _OUTN), lambda i: (i, 0, 0)),
        ],
        out_specs=pl.BlockSpec((_BR2, _OUTC, _OUTN), lambda i: (i, 0, 0)),
        out_shape=jax.ShapeDtypeStruct((_N, _OUTC, _OUTN), jnp.float32),
        input_output_aliases={3: 0},
        compiler_params=pltpu.CompilerParams(
            dimension_semantics=("arbitrary",),
        ),
    )(pooled, ct_w, b2, y1)


# speedup vs baseline: 4.4694x; 2.1688x over previous
import jax
import jax.numpy as jnp
from jax.experimental import pallas as pl
from jax.experimental.pallas import tpu as pltpu

_INC, _INN, _OUTC, _OUTN, _MAXD, _N = 128, 256, 128, 64, 4, 1024
_BN = 128


def _fgl_block(x_ref, w_ref, wm_ref, a_ref, ctw_ref, b_ref, out_ref):
    k_iota = jax.lax.broadcasted_iota(jnp.int32, (_OUTN, _INN), 1)
    a = a_ref[...]
    wm = wm_ref[...]
    ct = jnp.zeros((_OUTN, _INN), jnp.float32)
    for d in range(_MAXD):
        ct = ct + jnp.where(k_iota == a[:, d : d + 1], wm[:, d : d + 1], 0.0)

    xw = x_ref[...] * w_ref[...][None, :, :]
    pooled = jax.lax.dot_general(
        xw.reshape(_BN * _INC, _INN), ct,
        (((1,), (1,)), ((), ())),
        preferred_element_type=jnp.float32,
    ).reshape(_BN, _INC, _OUTN)

    ctw = ctw_ref[...]
    b = b_ref[...]
    for j in range(_BN):
        out_ref[j] = (
            jax.lax.dot_general(
                ctw, pooled[j], (((1,), (0,)), ((), ())),
                preferred_element_type=jnp.float32,
            )
            + b
        )


def kernel(x, weight, mask_weight, ct_w, ct_b, bias, A, mask):
    wm = (mask_weight * mask).reshape(_OUTN, _MAXD)
    b2 = bias + ct_b[:, None]
    grid = (_N // _BN,)
    return pl.pallas_call(
        _fgl_block,
        grid=grid,
        in_specs=[
            pl.BlockSpec((_BN, _INC, _INN), lambda i: (i, 0, 0)),
            pl.BlockSpec((_INC, _INN), lambda i: (0, 0)),
            pl.BlockSpec((_OUTN, _MAXD), lambda i: (0, 0)),
            pl.BlockSpec((_OUTN, _MAXD), lambda i: (0, 0)),
            pl.BlockSpec((_OUTC, _INC), lambda i: (0, 0)),
            pl.BlockSpec((_OUTC, _OUTN), lambda i: (0, 0)),
        ],
        out_specs=pl.BlockSpec((_BN, _OUTC, _OUTN), lambda i: (i, 0, 0)),
        out_shape=jax.ShapeDtypeStruct((_N, _OUTC, _OUTN), jnp.float32),
        compiler_params=pltpu.CompilerParams(
            dimension_semantics=("parallel",),
        ),
    )(x, weight, wm, A, ct_w, b2)
